# baseline (speedup 1.0000x reference)
import jax
import jax.numpy as jnp
from jax.experimental import pallas as pl

_K = 1024
_D = 64
_TILE = 512


def _vq_tile(x_ref, cbt_ref, cb_ref, csq_ref, q_ref, idx_ref):
    x = x_ref[...]
    xsq = jnp.sum(x * x, axis=1, keepdims=True)
    m = jnp.matmul(x, cbt_ref[...])
    dist = (xsq + csq_ref[...]) - 2.0 * m
    minval = jnp.min(dist, axis=1, keepdims=True)
    iota = jax.lax.broadcasted_iota(jnp.int32, dist.shape, 1)
    idx = jnp.min(jnp.where(dist == minval, iota, _K), axis=1, keepdims=True)
    onehot = (iota == idx).astype(jnp.float32)
    q_ref[...] = jax.lax.dot(onehot, cb_ref[...],
                             precision=jax.lax.Precision.HIGHEST)
    idx_ref[...] = idx


def kernel(inputs, codebook):
    input_shape = inputs.shape
    flat = inputs.reshape(-1, _D)
    n = flat.shape[0]
    csq = jnp.sum(codebook ** 2, axis=1)[None, :]
    cbt = codebook.T
    grid = (n // _TILE,)
    q, idx = pl.pallas_call(
        _vq_tile,
        grid=grid,
        in_specs=[
            pl.BlockSpec((_TILE, _D), lambda i: (i, 0)),
            pl.BlockSpec((_D, _K), lambda i: (0, 0)),
            pl.BlockSpec((_K, _D), lambda i: (0, 0)),
            pl.BlockSpec((1, _K), lambda i: (0, 0)),
        ],
        out_specs=[
            pl.BlockSpec((_TILE, _D), lambda i: (i, 0)),
            pl.BlockSpec((_TILE, 1), lambda i: (i, 0)),
        ],
        out_shape=[
            jax.ShapeDtypeStruct((n, _D), jnp.float32),
            jax.ShapeDtypeStruct((n, 1), jnp.int32),
        ],
    )(flat, cbt, codebook, csq)
    quantized = (flat + (q - flat)).reshape(input_shape)
    return (quantized, idx.reshape(-1))


# speedup vs baseline: 1.3424x; 1.3424x over previous
import functools

import jax
import jax.numpy as jnp
from jax.experimental import pallas as pl
from jax.experimental.pallas import tpu as pltpu
from jax.experimental.pallas import tpu_sc as plsc

_K = 1024
_D = 64
_TILE = 512
_CH = 96


def _vq_idx_tile(x_ref, cbt2_ref, csq_ref, idx_ref):
    x = x_ref[...]
    xsq = jnp.sum(x * x, axis=1, keepdims=True)
    m2 = jnp.matmul(x, cbt2_ref[...])
    dist = (xsq + csq_ref[...]) - m2
    minval = jnp.min(dist, axis=1, keepdims=True)
    iota = jax.lax.broadcasted_iota(jnp.int32, dist.shape, 1)
    idx_ref[...] = jnp.min(jnp.where(dist == minval, iota, _K), axis=1,
                           keepdims=True)


def _tc_argmin(flat, codebook):
    n = flat.shape[0]
    csq = jnp.sum(codebook ** 2, axis=1)[None, :]
    cbt2 = codebook.T + codebook.T
    return pl.pallas_call(
        _vq_idx_tile,
        grid=(n // _TILE,),
        in_specs=[
            pl.BlockSpec((_TILE, _D), lambda i: (i, 0)),
            pl.BlockSpec((_D, _K), lambda i: (0, 0)),
            pl.BlockSpec((1, _K), lambda i: (0, 0)),
        ],
        out_specs=pl.BlockSpec((_TILE, 1), lambda i: (i, 0)),
        out_shape=jax.ShapeDtypeStruct((n, 1), jnp.int32),
    )(flat, cbt2, csq)


def _sc_gather(codebook, idx, n):
    cb_pad = jnp.pad(codebook, ((0, 0), (0, 128 - _D)))
    info = plsc.get_sparse_core_info()
    nw = info.num_cores * info.num_subcores
    b_per_w = n // nw
    n_ch = b_per_w // _CH
    idx3d = idx.reshape(nw, n_ch, _CH)
    mesh = plsc.VectorSubcoreMesh(core_axis_name="c", subcore_axis_name="s")

    @functools.partial(
        pl.kernel, mesh=mesh,
        out_type=jax.ShapeDtypeStruct((n, 128), jnp.float32),
        scratch_types=[
            pltpu.VMEM((n_ch, _CH), jnp.int32),
            pltpu.VMEM((b_per_w, 128), jnp.float32),
            pltpu.SemaphoreType.DMA,
        ],
    )
    def gather_k(cb_hbm, idx_hbm, out_hbm, idx_v, rows_v, sem):
        wid = jax.lax.axis_index("s") * info.num_cores + jax.lax.axis_index("c")
        pltpu.sync_copy(idx_hbm.at[wid], idx_v)
        copies = [
            pltpu.async_copy(cb_hbm.at[idx_v.at[j]],
                             rows_v.at[pl.ds(j * _CH, _CH)], sem)
            for j in range(n_ch)
        ]
        for c in copies:
            c.wait()
        pltpu.sync_copy(rows_v, out_hbm.at[pl.ds(wid * b_per_w, b_per_w)])

    return gather_k(cb_pad, idx3d)[:, :_D]


def kernel(inputs, codebook):
    input_shape = inputs.shape
    flat = inputs.reshape(-1, _D)
    n = flat.shape[0]
    idx = _tc_argmin(flat, codebook)
    q = _sc_gather(codebook, idx.reshape(-1), n)
    quantized = (flat + (q - flat)).reshape(input_shape)
    return (quantized, idx.reshape(-1))


# speedup vs baseline: 1.4239x; 1.0607x over previous
import functools

import jax
import jax.numpy as jnp
from jax.experimental import pallas as pl
from jax.experimental.pallas import tpu as pltpu
from jax.experimental.pallas import tpu_sc as plsc

_K = 1024
_D = 64
_TILE = 512
_CH = 96


def _vq_idx_tile(x_ref, cbt2_ref, csq_ref, idx_ref):
    x = x_ref[...]
    xsq = jnp.sum(x * x, axis=1, keepdims=True)
    m2 = jnp.matmul(x, cbt2_ref[...])
    dist = (xsq + csq_ref[...]) - m2
    minval = jnp.min(dist, axis=1, keepdims=True)
    iota = jax.lax.broadcasted_iota(jnp.int32, dist.shape, 1)
    idx = jnp.min(jnp.where(dist == minval, iota, _K), axis=1)
    idx_ref[...] = idx.reshape(1, _TILE // 128, 128)


def _tc_argmin(flat, codebook):
    n = flat.shape[0]
    csq = jnp.sum(codebook ** 2, axis=1)[None, :]
    cbt2 = codebook.T + codebook.T
    return pl.pallas_call(
        _vq_idx_tile,
        grid=(n // _TILE,),
        in_specs=[
            pl.BlockSpec((_TILE, _D), lambda i: (i, 0)),
            pl.BlockSpec((_D, _K), lambda i: (0, 0)),
            pl.BlockSpec((1, _K), lambda i: (0, 0)),
        ],
        out_specs=pl.BlockSpec((1, _TILE // 128, 128), lambda i: (i, 0, 0)),
        out_shape=jax.ShapeDtypeStruct((n // _TILE, _TILE // 128, 128),
                                       jnp.int32),
        compiler_params=pltpu.CompilerParams(
            dimension_semantics=("parallel",)),
    )(flat, cbt2, csq)


def _sc_gather(codebook, idx, n):
    cb_pad = jnp.pad(codebook, ((0, 0), (0, 128 - _D)))
    info = plsc.get_sparse_core_info()
    nw = info.num_cores * info.num_subcores
    b_per_w = n // nw
    n_ch = b_per_w // _CH
    idx3d = idx.reshape(nw, n_ch, _CH)
    mesh = plsc.VectorSubcoreMesh(core_axis_name="c", subcore_axis_name="s")

    @functools.partial(
        pl.kernel, mesh=mesh,
        out_type=jax.ShapeDtypeStruct((n, 128), jnp.float32),
        scratch_types=[
            pltpu.VMEM((n_ch, _CH), jnp.int32),
            pltpu.VMEM((b_per_w, 128), jnp.float32),
            pltpu.SemaphoreType.DMA,
        ],
    )
    def gather_k(cb_hbm, idx_hbm, out_hbm, idx_v, rows_v, sem):
        wid = jax.lax.axis_index("s") * info.num_cores + jax.lax.axis_index("c")
        pltpu.sync_copy(idx_hbm.at[wid], idx_v)
        copies = [
            pltpu.async_copy(cb_hbm.at[idx_v.at[j]],
                             rows_v.at[pl.ds(j * _CH, _CH)], sem)
            for j in range(n_ch)
        ]
        for c in copies:
            c.wait()
        pltpu.sync_copy(rows_v, out_hbm.at[pl.ds(wid * b_per_w, b_per_w)])

    return gather_k(cb_pad, idx3d)[:, :_D]


def kernel(inputs, codebook):
    input_shape = inputs.shape
    flat = inputs.reshape(-1, _D)
    n = flat.shape[0]
    idx = _tc_argmin(flat, codebook)
    q = _sc_gather(codebook, idx.reshape(-1), n)
    quantized = (flat + (q - flat)).reshape(input_shape)
    return (quantized, idx.reshape(-1))
